# baseline (speedup 1.0000x reference)
import functools

import jax
import jax.numpy as jnp
from jax import lax
from jax.experimental import pallas as pl
from jax.experimental.pallas import tpu as pltpu
from jax.experimental.pallas import tpu_sc as plsc

CH, CW = 448, 448
TOP, LEFT = 32, 32
NC, NS = 2, 16
NW = NC * NS
N = 192
IPW = N // NW
RB = 64
CPI = CH // RB
T = IPW * CPI
P = T // 2

_mesh = plsc.VectorSubcoreMesh(core_axis_name="c", subcore_axis_name="s")


@functools.partial(
    pl.kernel,
    out_type=jax.ShapeDtypeStruct((N, CH, CW), jnp.float32),
    mesh=_mesh,
    scratch_types=[
        pltpu.VMEM((2, RB, 512), jnp.float32),
        pltpu.VMEM((2, RB, CW), jnp.float32),
        pltpu.SemaphoreType.DMA,
        pltpu.SemaphoreType.DMA,
        pltpu.SemaphoreType.DMA,
        pltpu.SemaphoreType.DMA,
    ],
    compiler_params=pltpu.CompilerParams(use_tc_tiling_on_sc=True),
)
def _sc_crop(in_hbm, out_hbm, bin_, bout, l0, l1, s0, s1):
    wid = lax.axis_index("s") * NC + lax.axis_index("c")
    base = wid * IPW
    lsem = (l0, l1)
    ssem = (s0, s1)

    def coords(c):
        return base + c // CPI, lax.rem(c, CPI)

    def in_copies(s, img, rb):
        row0 = TOP + rb * RB
        return [
            pltpu.make_async_copy(
                in_hbm.at[img, pl.ds(row0, RB), :],
                bin_.at[s],
                lsem[s],
            )
        ]

    def out_copies(s, img, rb):
        row0 = rb * RB
        return [
            pltpu.make_async_copy(
                bout.at[s],
                out_hbm.at[img, pl.ds(row0, RB), :],
                ssem[s],
            )
        ]

    def start(cps):
        for c in cps:
            c.start()

    def wait(cps):
        for c in cps:
            c.wait()

    def shuffle(s):
        def body(r2, carry):
            for u in range(2):
                r = 2 * r2 + u
                for k in range(CW // 16):
                    bout[s, r, pl.ds(k * 16, 16)] = (
                        bin_[s, r, pl.ds(LEFT + k * 16, 16)])
            return carry

        lax.fori_loop(0, RB // 2, body, 0)

    def stage(s, c, first, last):
        img, rb = coords(c)
        wait(in_copies(s, img, rb))
        if not first:
            wait(out_copies(s, img, rb))
        shuffle(s)
        start(out_copies(s, img, rb))
        if not last:
            img2, rb2 = coords(c + 2)
            start(in_copies(s, img2, rb2))

    img, rb = coords(0)
    start(in_copies(0, img, rb))
    img, rb = coords(1)
    start(in_copies(1, img, rb))

    stage(0, 0, True, False)
    stage(1, 1, True, False)

    def body(p, carry):
        stage(0, 2 * p, False, False)
        stage(1, 2 * p + 1, False, False)
        return carry

    lax.fori_loop(1, P - 1, body, 0)

    stage(0, T - 2, False, True)
    stage(1, T - 1, False, True)

    img, rb = coords(T - 2)
    wait(out_copies(0, img, rb))
    img, rb = coords(T - 1)
    wait(out_copies(1, img, rb))


def kernel(x):
    B, Tt, C, H, W = x.shape
    xf = x.reshape(N, H, W)
    out = _sc_crop(xf)
    return out.reshape(B, Tt * C, CH, CW)


# speedup vs baseline: 1.0229x; 1.0229x over previous
import functools

import jax
import jax.numpy as jnp
from jax import lax
from jax.experimental import pallas as pl
from jax.experimental.pallas import tpu as pltpu
from jax.experimental.pallas import tpu_sc as plsc

CH, CW = 448, 448
TOP, LEFT = 32, 32
NC, NS = 2, 16
NW = NC * NS
N = 192
IPW = N // NW
RB = 64
CPI = CH // RB
T = IPW * CPI
P = T // 2

_mesh = plsc.VectorSubcoreMesh(core_axis_name="c", subcore_axis_name="s")


@functools.partial(
    pl.kernel,
    out_type=jax.ShapeDtypeStruct((N, CH, CW), jnp.float32),
    mesh=_mesh,
    scratch_types=[
        pltpu.VMEM((2, RB, 512), jnp.float32),
        pltpu.VMEM((2, RB, CW), jnp.float32),
        pltpu.SemaphoreType.DMA,
        pltpu.SemaphoreType.DMA,
        pltpu.SemaphoreType.DMA,
        pltpu.SemaphoreType.DMA,
    ],
    compiler_params=pltpu.CompilerParams(use_tc_tiling_on_sc=True),
)
def _sc_crop(in_hbm, out_hbm, bin_, bout, l0, l1, s0, s1):
    wid = lax.axis_index("s") * NC + lax.axis_index("c")
    base = wid * IPW
    lsem = (l0, l1)
    ssem = (s0, s1)

    def coords(c):
        return base + c // CPI, lax.rem(c, CPI)

    def in_copies(s, img, rb):
        row0 = TOP + rb * RB
        return [
            pltpu.make_async_copy(
                in_hbm.at[img, pl.ds(row0, RB), :],
                bin_.at[s],
                lsem[s],
            )
        ]

    def out_copies(s, img, rb):
        row0 = rb * RB
        return [
            pltpu.make_async_copy(
                bout.at[s],
                out_hbm.at[img, pl.ds(row0, RB), :],
                ssem[s],
            )
        ]

    def start(cps):
        for c in cps:
            c.start()

    def wait(cps):
        for c in cps:
            c.wait()

    def shuffle(s):
        def body(r, carry):
            for k in range(CW // 16):
                bout[s, r, pl.ds(k * 16, 16)] = (
                    bin_[s, r, pl.ds(LEFT + k * 16, 16)])
            return carry

        lax.fori_loop(0, RB, body, 0)

    def stage(s, c, first, last):
        img, rb = coords(c)
        wait(in_copies(s, img, rb))
        if not first:
            wait(out_copies(s, img, rb))
        shuffle(s)
        start(out_copies(s, img, rb))
        if not last:
            img2, rb2 = coords(c + 2)
            start(in_copies(s, img2, rb2))

    img, rb = coords(0)
    start(in_copies(0, img, rb))
    img, rb = coords(1)
    start(in_copies(1, img, rb))

    stage(0, 0, True, False)
    stage(1, 1, True, False)

    def body(p, carry):
        stage(0, 2 * p, False, False)
        stage(1, 2 * p + 1, False, False)
        return carry

    lax.fori_loop(1, P - 1, body, 0)

    stage(0, T - 2, False, True)
    stage(1, T - 1, False, True)

    img, rb = coords(T - 2)
    wait(out_copies(0, img, rb))
    img, rb = coords(T - 1)
    wait(out_copies(1, img, rb))


def kernel(x):
    B, Tt, C, H, W = x.shape
    xf = x.reshape(N, H, W)
    out = _sc_crop(xf)
    return out.reshape(B, Tt * C, CH, CW)
